# Initial kernel scaffold; baseline (speedup 1.0000x reference)
import jax
import jax.numpy as jnp
from jax.experimental import pallas as pl


def kernel(x, conv0, conv1, fc0, fc1):
    raise NotImplementedError("write your pallas kernel here")


# speedup vs baseline: 56.3257x; 56.3257x over previous
import numpy as np
import jax
import jax.numpy as jnp
from jax.experimental import pallas as pl
from jax.experimental.pallas import tpu as pltpu


def _band0(conv0):
    d = np.arange(5)[:, None, None, None]
    i = np.arange(28)[None, :, None, None]
    P = np.arange(2)[None, None, :, None]
    p = np.arange(12)[None, None, None, :]
    M0 = jnp.asarray((i == 2 * p + P + d).astype(np.float32))
    A0 = jnp.einsum('rdc,diPp->riPpc', conv0[:, :, 0, :], M0)
    return A0.reshape(140, 768)


def _band1(conv1):
    d = np.arange(5)[:, None, None, None]
    w = np.arange(12)[None, :, None, None]
    P = np.arange(2)[None, None, :, None]
    p = np.arange(4)[None, None, None, :]
    M1 = jnp.asarray((w == 2 * p + P + d).astype(np.float32))
    A1 = jnp.einsum('rdio,dwPp->rwiPpo', conv1, M1)
    return A1.reshape(1920, 512)


def _fused_kernel(x_ref, a0_ref, a1_ref, w1_ref, w2_ref, o_ref, f1_ref, f2_ref):
    f32 = jnp.float32
    a0 = a0_ref[...]
    a1 = a1_ref[...]
    for ph in range(12):
        s0 = x_ref[:, 56 * ph: 56 * ph + 140]
        s1 = x_ref[:, 56 * ph + 28: 56 * ph + 168]
        r0 = jnp.dot(s0, a0, preferred_element_type=f32)
        r1 = jnp.dot(s1, a0, preferred_element_type=f32)
        v = jnp.maximum(jnp.maximum(r0, r1), 0.0)
        f1_ref[:, ph * 384:(ph + 1) * 384] = jnp.maximum(v[:, :384], v[:, 384:])
    for ph in range(4):
        base = 2 * ph * 384
        t0 = jnp.dot(f1_ref[:, base: base + 1920], a1, preferred_element_type=f32)
        t1 = jnp.dot(f1_ref[:, base + 384: base + 2304], a1, preferred_element_type=f32)
        v = jnp.maximum(jnp.maximum(t0, t1), 0.0)
        f2_ref[:, ph * 256:(ph + 1) * 256] = jnp.maximum(v[:, :256], v[:, 256:])
    h = jnp.maximum(jnp.dot(f2_ref[...], w1_ref[...], preferred_element_type=f32), 0.0)
    y = jnp.dot(h, w2_ref[...], preferred_element_type=f32)
    m = jnp.max(y, axis=-1, keepdims=True)
    s = y - m
    lse = jnp.log(jnp.sum(jnp.exp(s), axis=-1, keepdims=True))
    o_ref[...] = (s - lse).astype(o_ref.dtype)


def kernel(x, conv0, conv1, fc0, fc1):
    B = x.shape[0]
    T = 256 if B % 256 == 0 else (128 if B % 128 == 0 else B)
    n_hid = fc0.shape[1]
    x2 = x.reshape(B, 28 * 28)
    A0 = _band0(conv0)
    A1 = _band1(conv1)
    W1 = fc0.reshape(64, 4, 4, n_hid).transpose(1, 2, 0, 3).reshape(1024, n_hid)
    return pl.pallas_call(
        _fused_kernel,
        out_shape=jax.ShapeDtypeStruct((B, 10), x.dtype),
        grid=(B // T,),
        in_specs=[
            pl.BlockSpec((T, 784), lambda i: (i, 0)),
            pl.BlockSpec((140, 768), lambda i: (0, 0)),
            pl.BlockSpec((1920, 512), lambda i: (0, 0)),
            pl.BlockSpec((1024, n_hid), lambda i: (0, 0)),
            pl.BlockSpec((n_hid, 10), lambda i: (0, 0)),
        ],
        out_specs=pl.BlockSpec((T, 10), lambda i: (i, 0)),
        scratch_shapes=[
            pltpu.VMEM((T, 4608), jnp.float32),
            pltpu.VMEM((T, 1024), jnp.float32),
        ],
        compiler_params=pltpu.CompilerParams(dimension_semantics=("parallel",)),
    )(x2, A0, A1, W1, fc1)


# speedup vs baseline: 60.7710x; 1.0789x over previous
import numpy as np
import jax
import jax.numpy as jnp
from jax.experimental import pallas as pl
from jax.experimental.pallas import tpu as pltpu


def _band0(conv0):
    d = np.arange(5)[:, None, None, None]
    i = np.arange(28)[None, :, None, None]
    P = np.arange(2)[None, None, :, None]
    p = np.arange(12)[None, None, None, :]
    M0 = jnp.asarray((i == 2 * p + P + d).astype(np.float32))
    A0 = jnp.einsum('rdc,diPp->riPpc', conv0[:, :, 0, :], M0)
    return A0.reshape(140, 768)


def _band1(conv1):
    d = np.arange(5)[:, None, None, None]
    w = np.arange(12)[None, :, None, None]
    P = np.arange(2)[None, None, :, None]
    p = np.arange(4)[None, None, None, :]
    M1 = jnp.asarray((w == 2 * p + P + d).astype(np.float32))
    A1 = jnp.einsum('rdio,dwPp->rwiPpo', conv1, M1)
    return A1.reshape(1920, 512)


def _fused_kernel(x_ref, a0_ref, a1_ref, w1_ref, w2_ref, o_ref, f1_ref, f2_ref):
    f32 = jnp.float32
    a0 = a0_ref[...]
    a1 = a1_ref[...]
    for ph in range(12):
        s0 = x_ref[:, 56 * ph: 56 * ph + 140]
        s1 = x_ref[:, 56 * ph + 28: 56 * ph + 168]
        r0 = jnp.dot(s0, a0, preferred_element_type=f32)
        r1 = jnp.dot(s1, a0, preferred_element_type=f32)
        v = jnp.maximum(jnp.maximum(r0, r1), 0.0)
        f1_ref[:, ph * 384:(ph + 1) * 384] = jnp.maximum(
            v[:, :384], v[:, 384:]).astype(f1_ref.dtype)
    for ph in range(4):
        base = 2 * ph * 384
        t0 = jnp.dot(f1_ref[:, base: base + 1920], a1, preferred_element_type=f32)
        t1 = jnp.dot(f1_ref[:, base + 384: base + 2304], a1, preferred_element_type=f32)
        v = jnp.maximum(jnp.maximum(t0, t1), 0.0)
        f2_ref[:, ph * 256:(ph + 1) * 256] = jnp.maximum(
            v[:, :256], v[:, 256:]).astype(f2_ref.dtype)
    h = jnp.maximum(jnp.dot(f2_ref[...], w1_ref[...], preferred_element_type=f32), 0.0)
    y = jnp.dot(h.astype(w2_ref.dtype), w2_ref[...], preferred_element_type=f32)
    m = jnp.max(y, axis=-1, keepdims=True)
    s = y - m
    lse = jnp.log(jnp.sum(jnp.exp(s), axis=-1, keepdims=True))
    o_ref[...] = (s - lse).astype(o_ref.dtype)


def kernel(x, conv0, conv1, fc0, fc1):
    B = x.shape[0]
    T = 256 if B % 256 == 0 else (128 if B % 128 == 0 else B)
    n_hid = fc0.shape[1]
    cd = jnp.bfloat16
    x2 = x.reshape(B, 28 * 28).astype(cd)
    A0 = _band0(conv0).astype(cd)
    A1 = _band1(conv1).astype(cd)
    W1 = fc0.reshape(64, 4, 4, n_hid).transpose(1, 2, 0, 3).reshape(1024, n_hid).astype(cd)
    return pl.pallas_call(
        _fused_kernel,
        out_shape=jax.ShapeDtypeStruct((B, 10), x.dtype),
        grid=(B // T,),
        in_specs=[
            pl.BlockSpec((T, 784), lambda i: (i, 0)),
            pl.BlockSpec((140, 768), lambda i: (0, 0)),
            pl.BlockSpec((1920, 512), lambda i: (0, 0)),
            pl.BlockSpec((1024, n_hid), lambda i: (0, 0)),
            pl.BlockSpec((n_hid, 10), lambda i: (0, 0)),
        ],
        out_specs=pl.BlockSpec((T, 10), lambda i: (i, 0)),
        scratch_shapes=[
            pltpu.VMEM((T, 4608), cd),
            pltpu.VMEM((T, 1024), cd),
        ],
        compiler_params=pltpu.CompilerParams(dimension_semantics=("parallel",)),
    )(x2, A0, A1, W1, fc1.astype(cd))
